# baseline (speedup 1.0000x reference)
import functools

import jax
import jax.numpy as jnp
from jax.experimental import pallas as pl
from jax.experimental.pallas import tpu as pltpu

_BLK = 512


def _body(x_ref, logit_ref, ema_ref, out_ref):
    x = x_ref[...]
    blk, c = x.shape
    m = jnp.max(x, axis=1, keepdims=True)
    iota = jax.lax.broadcasted_iota(jnp.int32, (blk, c), 1)
    idx = jnp.min(jnp.where(x == m, iota, c), axis=1, keepdims=True)
    onehot = (iota == idx).astype(jnp.float32)
    thr = jax.lax.dot_general(
        onehot, ema_ref[...],
        dimension_numbers=(((1,), (1,)), ((), ())),
        preferred_element_type=jnp.float32,
        precision=jax.lax.Precision.HIGHEST,
    )
    diff = jnp.maximum(thr - logit_ref[...], 0.0)
    out_ref[...] = diff * diff


def kernel(gen_logits, gen_classes, ema_real):
    b, c = gen_classes.shape
    grid = b // _BLK
    return pl.pallas_call(
        _body,
        grid=(grid,),
        in_specs=[
            pl.BlockSpec((_BLK, c), lambda i: (i, 0)),
            pl.BlockSpec((_BLK, 1), lambda i: (i, 0)),
            pl.BlockSpec((1, c), lambda i: (0, 0)),
        ],
        out_specs=pl.BlockSpec((_BLK, 1), lambda i: (i, 0)),
        out_shape=jax.ShapeDtypeStruct((b, 1), jnp.float32),
        compiler_params=pltpu.CompilerParams(
            dimension_semantics=("arbitrary",),
        ),
    )(gen_classes, gen_logits, ema_real.reshape(1, c))


# speedup vs baseline: 1.0318x; 1.0318x over previous
import functools

import jax
import jax.numpy as jnp
from jax.experimental import pallas as pl
from jax.experimental.pallas import tpu as pltpu

_BLK = 512


def _body(x_ref, logit_ref, ema_ref, out_ref):
    x = x_ref[...]
    blk, c = x.shape
    m = jnp.max(x, axis=1, keepdims=True)
    iota_f = jax.lax.broadcasted_iota(jnp.int32, (blk, c), 1).astype(jnp.float32)
    idxf = jnp.min(jnp.where(x == m, iota_f, 1024.0), axis=1, keepdims=True)
    ema_b = jnp.broadcast_to(ema_ref[...], (blk, c))
    thr = jnp.max(jnp.where(iota_f == idxf, ema_b, -jnp.inf), axis=1,
                  keepdims=True)
    diff = jnp.maximum(thr - logit_ref[...], 0.0)
    out_ref[...] = diff * diff


def kernel(gen_logits, gen_classes, ema_real):
    b, c = gen_classes.shape
    grid = b // _BLK
    return pl.pallas_call(
        _body,
        grid=(grid,),
        in_specs=[
            pl.BlockSpec((_BLK, c), lambda i: (i, 0)),
            pl.BlockSpec((_BLK, 1), lambda i: (i, 0)),
            pl.BlockSpec((1, c), lambda i: (0, 0)),
        ],
        out_specs=pl.BlockSpec((_BLK, 1), lambda i: (i, 0)),
        out_shape=jax.ShapeDtypeStruct((b, 1), jnp.float32),
        compiler_params=pltpu.CompilerParams(
            dimension_semantics=("arbitrary",),
        ),
    )(gen_classes, gen_logits, ema_real.reshape(1, c))
